# Initial kernel scaffold; baseline (speedup 1.0000x reference)
import jax
import jax.numpy as jnp
from jax.experimental import pallas as pl


def kernel(entity_emb, edge_index, edge_type):
    raise NotImplementedError("write your pallas kernel here")


# speedup vs baseline: 2.8234x; 2.8234x over previous
import functools

import jax
import jax.numpy as jnp
from jax import lax
from jax.experimental import pallas as pl
from jax.experimental.pallas import tpu as pltpu
from jax.experimental.pallas import tpu_sc as plsc

N_NODES_C = 10000
N_REL_C = 4
DIM_C = 128
N_EDGES_C = 320000

CHUNK = 1280


@functools.lru_cache(maxsize=None)
def _build(n_nodes, n_rel, dim, n_edges):
    info = plsc.get_sparse_core_info()
    nc, ns = info.num_cores, info.num_subcores
    nw = nc * ns
    assert dim % nw == 0
    dpw = dim // nw
    slice_words = n_nodes * dpw
    table_words = n_nodes * n_rel
    n_chunks = n_edges // CHUNK
    assert n_chunks * CHUNK == n_edges

    mesh = plsc.VectorSubcoreMesh(core_axis_name="c", subcore_axis_name="s")

    @functools.partial(
        pl.kernel,
        out_type=jax.ShapeDtypeStruct((nw, slice_words), jnp.float32),
        mesh=mesh,
        compiler_params=pltpu.CompilerParams(needs_layout_passes=False),
        scratch_types=[
            pltpu.VMEM((slice_words,), jnp.float32),
            pltpu.VMEM((slice_words,), jnp.float32),
            pltpu.VMEM((table_words,), jnp.float32),
            pltpu.VMEM((CHUNK,), jnp.int32),
            pltpu.VMEM((CHUNK,), jnp.int32),
            pltpu.VMEM((CHUNK,), jnp.int32),
        ],
    )
    def k(emb_hbm, src_hbm, dst_hbm, typ_hbm, out_hbm,
          emb_v, out_v, cnt_v, sbuf, dbuf, tbuf):
        wid = lax.axis_index("s") * nc + lax.axis_index("c")

        pltpu.sync_copy(emb_hbm.at[wid], emb_v)

        zf = jnp.zeros((16,), jnp.float32)

        def zero_body(i, carry):
            out_v[pl.ds(i * 16, 16)] = zf
            return carry

        lax.fori_loop(0, slice_words // 16, zero_body, 0)

        def zero_cnt(i, carry):
            cnt_v[pl.ds(i * 16, 16)] = zf
            return carry

        lax.fori_loop(0, table_words // 16, zero_cnt, 0)

        ones = jnp.ones((16,), jnp.float32)

        def count_chunk(g, carry):
            pltpu.sync_copy(dst_hbm.at[pl.ds(g * CHUNK, CHUNK)], dbuf)
            pltpu.sync_copy(typ_hbm.at[pl.ds(g * CHUNK, CHUNK)], tbuf)

            def inner(i, c2):
                d16 = dbuf[pl.ds(i * 16, 16)]
                t16 = tbuf[pl.ds(i * 16, 16)]
                plsc.addupdate_scatter(cnt_v, [d16 * n_rel + t16], ones)
                return c2

            lax.fori_loop(0, CHUNK // 16, inner, 0)
            return carry

        lax.fori_loop(0, n_chunks, count_chunk, 0)

        def conv(i, carry):
            c16 = cnt_v[pl.ds(i * 16, 16)]
            cnt_v[pl.ds(i * 16, 16)] = 1.0 / jnp.maximum(c16, 1.0)
            return carry

        lax.fori_loop(0, table_words // 16, conv, 0)

        def accum_chunk(g, carry):
            pltpu.sync_copy(src_hbm.at[pl.ds(g * CHUNK, CHUNK)], sbuf)
            pltpu.sync_copy(dst_hbm.at[pl.ds(g * CHUNK, CHUNK)], dbuf)
            pltpu.sync_copy(typ_hbm.at[pl.ds(g * CHUNK, CHUNK)], tbuf)

            def inner(i, c2):
                s16 = sbuf[pl.ds(i * 16, 16)]
                d16 = dbuf[pl.ds(i * 16, 16)]
                t16 = tbuf[pl.ds(i * 16, 16)]
                scale = plsc.load_gather(cnt_v, [d16 * n_rel + t16])
                sbase = s16 * dpw
                dbase = d16 * dpw
                for d in range(dpw):
                    vals = plsc.load_gather(emb_v, [sbase + d])
                    plsc.addupdate_scatter(out_v, [dbase + d], vals * scale)
                return c2

            lax.fori_loop(0, CHUNK // 16, inner, 0)
            return carry

        lax.fori_loop(0, n_chunks, accum_chunk, 0)

        pltpu.sync_copy(out_v, out_hbm.at[wid])

    return k, nw, dpw


def kernel(entity_emb, edge_index, edge_type):
    n_nodes, dim = entity_emb.shape
    n_edges = edge_type.shape[0]
    k, nw, dpw = _build(n_nodes, N_REL_C, dim, n_edges)
    emb_t = (entity_emb.reshape(n_nodes, nw, dpw)
             .transpose(1, 0, 2).reshape(nw, n_nodes * dpw))
    out_t = k(emb_t, edge_index[0], edge_index[1], edge_type)
    return (out_t.reshape(nw, n_nodes, dpw)
            .transpose(1, 0, 2).reshape(n_nodes, dim))


# speedup vs baseline: 3.9993x; 1.4165x over previous
import functools

import jax
import jax.numpy as jnp
from jax import lax
from jax.experimental import pallas as pl
from jax.experimental.pallas import tpu as pltpu
from jax.experimental.pallas import tpu_sc as plsc

N_REL_C = 4

CHUNK = 1280
UNROLL = 8


@functools.lru_cache(maxsize=None)
def _build(n_nodes, n_rel, dim, n_edges):
    info = plsc.get_sparse_core_info()
    nc, ns = info.num_cores, info.num_subcores
    nw = nc * ns
    assert dim % nw == 0
    dpw = dim // nw
    slice_words = n_nodes * dpw
    table_words = n_nodes * n_rel
    n_chunks = n_edges // CHUNK
    assert n_chunks * CHUNK == n_edges and n_chunks % 2 == 0
    ipc = CHUNK // 16

    mesh = plsc.VectorSubcoreMesh(core_axis_name="c", subcore_axis_name="s")

    @functools.partial(
        pl.kernel,
        out_type=jax.ShapeDtypeStruct((nw, slice_words), jnp.float32),
        mesh=mesh,
        compiler_params=pltpu.CompilerParams(needs_layout_passes=False),
        scratch_types=[
            pltpu.VMEM((slice_words,), jnp.float32),
            pltpu.VMEM((slice_words,), jnp.float32),
            pltpu.VMEM((table_words,), jnp.float32),
            pltpu.VMEM((2, CHUNK), jnp.int32),
            pltpu.VMEM((2, CHUNK), jnp.int32),
            pltpu.VMEM((2, CHUNK), jnp.int32),
            pltpu.SemaphoreType.DMA,
        ],
    )
    def k(emb_hbm, src_hbm, dst_hbm, typ_hbm, out_hbm,
          emb_v, out_v, cnt_v, sbuf, dbuf, tbuf, sem):
        wid = lax.axis_index("s") * nc + lax.axis_index("c")

        pltpu.sync_copy(emb_hbm.at[wid], emb_v)

        zf = jnp.zeros((16,), jnp.float32)

        @plsc.parallel_loop(0, slice_words // 16, unroll=UNROLL)
        def _(i):
            out_v[pl.ds(i * 16, 16)] = zf

        @plsc.parallel_loop(0, table_words // 16, unroll=UNROLL)
        def _(i):
            cnt_v[pl.ds(i * 16, 16)] = zf

        ones = jnp.ones((16,), jnp.float32)

        def count_chunk(g, carry):
            b = 0
            pltpu.sync_copy(dst_hbm.at[pl.ds(g * CHUNK, CHUNK)], dbuf.at[b])
            pltpu.sync_copy(typ_hbm.at[pl.ds(g * CHUNK, CHUNK)], tbuf.at[b])

            @plsc.parallel_loop(0, ipc, unroll=UNROLL)
            def _(i):
                d16 = dbuf[b, pl.ds(i * 16, 16)]
                t16 = tbuf[b, pl.ds(i * 16, 16)]
                plsc.addupdate_scatter(cnt_v, [d16 * n_rel + t16], ones)

            return carry

        lax.fori_loop(0, n_chunks, count_chunk, 0)

        @plsc.parallel_loop(0, table_words // 16, unroll=UNROLL)
        def _(i):
            c16 = cnt_v[pl.ds(i * 16, 16)]
            cnt_v[pl.ds(i * 16, 16)] = 1.0 / jnp.maximum(c16, 1.0)

        def accum_chunk(g, carry):
            b = 0
            pltpu.sync_copy(src_hbm.at[pl.ds(g * CHUNK, CHUNK)], sbuf.at[b])
            pltpu.sync_copy(dst_hbm.at[pl.ds(g * CHUNK, CHUNK)], dbuf.at[b])
            pltpu.sync_copy(typ_hbm.at[pl.ds(g * CHUNK, CHUNK)], tbuf.at[b])

            @plsc.parallel_loop(0, ipc, unroll=UNROLL)
            def _(i):
                s16 = sbuf[b, pl.ds(i * 16, 16)]
                d16 = dbuf[b, pl.ds(i * 16, 16)]
                t16 = tbuf[b, pl.ds(i * 16, 16)]
                scale = plsc.load_gather(cnt_v, [d16 * n_rel + t16])
                sbase = s16 * dpw
                dbase = d16 * dpw
                for d in range(dpw):
                    vals = plsc.load_gather(emb_v, [sbase + d])
                    plsc.addupdate_scatter(out_v, [dbase + d],
                                           vals * scale)

            return carry

        lax.fori_loop(0, n_chunks, accum_chunk, 0)

        pltpu.sync_copy(out_v, out_hbm.at[wid])

    return k, nw, dpw


def kernel(entity_emb, edge_index, edge_type):
    n_nodes, dim = entity_emb.shape
    n_edges = edge_type.shape[0]
    k, nw, dpw = _build(n_nodes, N_REL_C, dim, n_edges)
    emb_t = (entity_emb.reshape(n_nodes, nw, dpw)
             .transpose(1, 0, 2).reshape(nw, n_nodes * dpw))
    out_t = k(emb_t, edge_index[0], edge_index[1], edge_type)
    return (out_t.reshape(nw, n_nodes, dpw)
            .transpose(1, 0, 2).reshape(n_nodes, dim))


# speedup vs baseline: 7.3801x; 1.8453x over previous
import functools

import jax
import jax.numpy as jnp
from jax import lax
from jax.experimental import pallas as pl
from jax.experimental.pallas import tpu as pltpu
from jax.experimental.pallas import tpu_sc as plsc

N_REL_C = 4

CHUNK = 1280
UNROLL = 8


@functools.lru_cache(maxsize=None)
def _build(n_nodes, n_rel, dim, n_edges):
    info = plsc.get_sparse_core_info()
    nc, ns = info.num_cores, info.num_subcores
    nw = nc * ns
    assert dim % nw == 0
    dpw = dim // nw
    slice_words = n_nodes * dpw
    table_words = n_nodes * n_rel
    n_chunks = n_edges // CHUNK
    assert n_chunks * CHUNK == n_edges and n_chunks % 2 == 0
    ipc = CHUNK // 16

    mesh = plsc.VectorSubcoreMesh(core_axis_name="c", subcore_axis_name="s")

    @functools.partial(
        pl.kernel,
        out_type=jax.ShapeDtypeStruct((nw, slice_words), jnp.float32),
        mesh=mesh,
        compiler_params=pltpu.CompilerParams(needs_layout_passes=False),
        scratch_types=[
            pltpu.VMEM((slice_words,), jnp.float32),
            pltpu.VMEM((slice_words,), jnp.float32),
            pltpu.VMEM((table_words,), jnp.float32),
            pltpu.VMEM((2, 3 * CHUNK), jnp.int32),
            pltpu.SemaphoreType.DMA,
            pltpu.SemaphoreType.DMA,
            pltpu.SemaphoreType.DMA,
        ],
    )
    def k(emb_hbm, edges_hbm, out_hbm,
          emb_v, out_v, cnt_v, ebuf, sem0, sem1, sem_e):
        wid = lax.axis_index("s") * nc + lax.axis_index("c")
        sems = (sem0, sem1)

        emb_cp = pltpu.async_copy(emb_hbm.at[wid], emb_v, sem_e)

        zf = jnp.zeros((16,), jnp.float32)

        @plsc.parallel_loop(0, slice_words // 16, unroll=UNROLL)
        def _(i):
            out_v[pl.ds(i * 16, 16)] = zf

        @plsc.parallel_loop(0, table_words // 16, unroll=UNROLL)
        def _(i):
            cnt_v[pl.ds(i * 16, 16)] = zf

        ones = jnp.ones((16,), jnp.float32)

        def start(g, slot):
            pltpu.async_copy(edges_hbm.at[g], ebuf.at[slot], sems[slot])

        def wait(g, slot):
            pltpu.make_async_copy(edges_hbm.at[g], ebuf.at[slot],
                                  sems[slot]).wait()

        def streamed(compute):
            start(0, 0)
            start(1, 1)

            def pair(g2, carry):
                g = g2 * 2
                wait(g, 0)
                compute(0)
                start(g + 2, 0)
                wait(g + 1, 1)
                compute(1)
                start(g + 3, 1)
                return carry

            lax.fori_loop(0, n_chunks // 2 - 1, pair, 0)
            wait(n_chunks - 2, 0)
            compute(0)
            wait(n_chunks - 1, 1)
            compute(1)

        def count_chunk(slot):
            @plsc.parallel_loop(0, ipc, unroll=UNROLL)
            def _(i):
                d16 = ebuf[slot, pl.ds(CHUNK + i * 16, 16)]
                t16 = ebuf[slot, pl.ds(2 * CHUNK + i * 16, 16)]
                plsc.addupdate_scatter(cnt_v, [d16 * n_rel + t16], ones)

        streamed(count_chunk)

        @plsc.parallel_loop(0, table_words // 16, unroll=UNROLL)
        def _(i):
            c16 = cnt_v[pl.ds(i * 16, 16)]
            cnt_v[pl.ds(i * 16, 16)] = 1.0 / jnp.maximum(c16, 1.0)

        emb_cp.wait()

        def accum_chunk(slot):
            @plsc.parallel_loop(0, ipc, unroll=UNROLL)
            def _(i):
                s16 = ebuf[slot, pl.ds(i * 16, 16)]
                d16 = ebuf[slot, pl.ds(CHUNK + i * 16, 16)]
                t16 = ebuf[slot, pl.ds(2 * CHUNK + i * 16, 16)]
                scale = plsc.load_gather(cnt_v, [d16 * n_rel + t16])
                sbase = s16 * dpw
                dbase = d16 * dpw
                for d in range(dpw):
                    vals = plsc.load_gather(emb_v, [sbase + d])
                    plsc.addupdate_scatter(out_v, [dbase + d],
                                           vals * scale)

        streamed(accum_chunk)

        pltpu.sync_copy(out_v, out_hbm.at[wid])

    return k, nw, dpw, n_chunks


def kernel(entity_emb, edge_index, edge_type):
    n_nodes, dim = entity_emb.shape
    n_edges = edge_type.shape[0]
    k, nw, dpw, n_chunks = _build(n_nodes, N_REL_C, dim, n_edges)
    emb_t = (entity_emb.reshape(n_nodes, nw, dpw)
             .transpose(1, 0, 2).reshape(nw, n_nodes * dpw))
    edges = jnp.stack([edge_index[0].reshape(n_chunks, CHUNK),
                       edge_index[1].reshape(n_chunks, CHUNK),
                       edge_type.reshape(n_chunks, CHUNK)],
                      axis=1).reshape(n_chunks, 3 * CHUNK)
    out_t = k(emb_t, edges)
    return (out_t.reshape(nw, n_nodes, dpw)
            .transpose(1, 0, 2).reshape(n_nodes, dim))


# speedup vs baseline: 8.6090x; 1.1665x over previous
import functools

import jax
import jax.numpy as jnp
from jax import lax
from jax.experimental import pallas as pl
from jax.experimental.pallas import tpu as pltpu
from jax.experimental.pallas import tpu_sc as plsc

N_REL_C = 4

CHUNK = 1280
UNROLL = 16


@functools.lru_cache(maxsize=None)
def _build(n_nodes, n_rel, dim, n_edges):
    info = plsc.get_sparse_core_info()
    nc, ns = info.num_cores, info.num_subcores
    nw = nc * ns
    assert dim % nw == 0
    dpw = dim // nw
    slice_words = n_nodes * dpw
    table_words = n_nodes * n_rel
    n_chunks = n_edges // CHUNK
    assert n_chunks * CHUNK == n_edges and n_chunks % 2 == 0
    ipc = CHUNK // 16

    mesh = plsc.VectorSubcoreMesh(core_axis_name="c", subcore_axis_name="s")

    @functools.partial(
        pl.kernel,
        out_type=jax.ShapeDtypeStruct((nw, slice_words), jnp.float32),
        mesh=mesh,
        compiler_params=pltpu.CompilerParams(needs_layout_passes=False),
        scratch_types=[
            pltpu.VMEM((slice_words,), jnp.float32),
            pltpu.VMEM((slice_words,), jnp.float32),
            pltpu.VMEM((table_words,), jnp.float32),
            pltpu.VMEM((2, 3 * CHUNK), jnp.int32),
            pltpu.SemaphoreType.DMA,
            pltpu.SemaphoreType.DMA,
            pltpu.SemaphoreType.DMA,
        ],
    )
    def k(emb_hbm, edges_hbm, out_hbm,
          emb_v, out_v, cnt_v, ebuf, sem0, sem1, sem_e):
        wid = lax.axis_index("s") * nc + lax.axis_index("c")
        sems = (sem0, sem1)

        emb_cp = pltpu.async_copy(emb_hbm.at[wid], emb_v, sem_e)

        zf = jnp.zeros((16,), jnp.float32)

        @plsc.parallel_loop(0, slice_words // 16, unroll=UNROLL)
        def _(i):
            out_v[pl.ds(i * 16, 16)] = zf

        @plsc.parallel_loop(0, table_words // 16, unroll=UNROLL)
        def _(i):
            cnt_v[pl.ds(i * 16, 16)] = zf

        ones = jnp.ones((16,), jnp.float32)

        def start(g, slot):
            pltpu.async_copy(edges_hbm.at[g], ebuf.at[slot], sems[slot])

        def wait(g, slot):
            pltpu.make_async_copy(edges_hbm.at[g], ebuf.at[slot],
                                  sems[slot]).wait()

        def streamed(compute):
            start(0, 0)
            start(1, 1)

            def pair(g2, carry):
                g = g2 * 2
                wait(g, 0)
                compute(0)
                start(g + 2, 0)
                wait(g + 1, 1)
                compute(1)
                start(g + 3, 1)
                return carry

            lax.fori_loop(0, n_chunks // 2 - 1, pair, 0)
            wait(n_chunks - 2, 0)
            compute(0)
            wait(n_chunks - 1, 1)
            compute(1)

        def count_chunk(slot):
            @plsc.parallel_loop(0, ipc, unroll=UNROLL)
            def _(i):
                d16 = ebuf[slot, pl.ds(CHUNK + i * 16, 16)]
                t16 = ebuf[slot, pl.ds(2 * CHUNK + i * 16, 16)]
                plsc.addupdate_scatter(cnt_v, [d16 * n_rel + t16], ones)

        streamed(count_chunk)

        @plsc.parallel_loop(0, table_words // 16, unroll=UNROLL)
        def _(i):
            c16 = cnt_v[pl.ds(i * 16, 16)]
            cnt_v[pl.ds(i * 16, 16)] = 1.0 / jnp.maximum(c16, 1.0)

        emb_cp.wait()

        def accum_chunk(slot):
            @plsc.parallel_loop(0, ipc, unroll=UNROLL)
            def _(i):
                s16 = ebuf[slot, pl.ds(i * 16, 16)]
                d16 = ebuf[slot, pl.ds(CHUNK + i * 16, 16)]
                t16 = ebuf[slot, pl.ds(2 * CHUNK + i * 16, 16)]
                scale = plsc.load_gather(cnt_v, [d16 * n_rel + t16])
                sbase = s16 * dpw
                dbase = d16 * dpw
                for d in range(dpw):
                    vals = plsc.load_gather(emb_v, [sbase + d])
                    plsc.addupdate_scatter(out_v, [dbase + d],
                                           vals * scale)

        streamed(accum_chunk)

        pltpu.sync_copy(out_v, out_hbm.at[wid])

    return k, nw, dpw, n_chunks


def kernel(entity_emb, edge_index, edge_type):
    n_nodes, dim = entity_emb.shape
    n_edges = edge_type.shape[0]
    k, nw, dpw, n_chunks = _build(n_nodes, N_REL_C, dim, n_edges)
    emb_t = (entity_emb.reshape(n_nodes, nw, dpw)
             .transpose(1, 0, 2).reshape(nw, n_nodes * dpw))
    edges = jnp.stack([edge_index[0].reshape(n_chunks, CHUNK),
                       edge_index[1].reshape(n_chunks, CHUNK),
                       edge_type.reshape(n_chunks, CHUNK)],
                      axis=1).reshape(n_chunks, 3 * CHUNK)
    out_t = k(emb_t, edges)
    return (out_t.reshape(nw, n_nodes, dpw)
            .transpose(1, 0, 2).reshape(n_nodes, dim))
